# baseline (speedup 1.0000x reference)
import functools
import math

import jax
from jax import lax
import jax.numpy as jnp
from jax.experimental import pallas as pl
from jax.experimental.pallas import tpu as pltpu
from jax.experimental.pallas import tpu_sc as plsc

_IMG = 800.0
_N = 20000
_NPAD = 20480
_ROWS = _NPAD // 128
_K = 2000
_KPAD = 2048
_OUT = 1000
_THRESH = 0.7
_MIN_SIZE = 0.001
_LOG_MAX = math.log(1000.0 / 16)
_IMIN = -(2 ** 31)
_NWORK = 32
_RPW = _ROWS // _NWORK


def _sortkey(x):
    b = lax.bitcast_convert_type(x, jnp.int32)
    return jnp.where(b >= 0, b, jnp.bitwise_xor(~b, jnp.int32(_IMIN)))


def _prep_body(ax0, ay0, ax1, ay1, rdx, rdy, rdw, rdh, sc,
               bx0, by0, bx1, by1, ms, pos):
    width = ax1[...] - ax0[...]
    height = ay1[...] - ay0[...]
    cx = ax0[...] + width / 2
    cy = ay0[...] + height / 2
    dw = jnp.minimum(rdw[...], _LOG_MAX)
    dh = jnp.minimum(rdh[...], _LOG_MAX)
    px = cx + rdx[...] * width
    py = cy + rdy[...] * height
    pw = jnp.exp(dw) * width
    ph = jnp.exp(dh) * height
    x0 = jnp.clip(px - 0.5 * pw, 0.0, _IMG)
    y0 = jnp.clip(py - 0.5 * ph, 0.0, _IMG)
    x1 = jnp.clip(px + 0.5 * pw, 0.0, _IMG)
    y1 = jnp.clip(py + 0.5 * ph, 0.0, _IMG)
    valid = ((x1 - x0) >= _MIN_SIZE) & ((y1 - y0) >= _MIN_SIZE)
    s = jnp.where(valid, sc[...], -jnp.inf)
    bx0[...] = x0
    by0[...] = y0
    bx1[...] = x1
    by1[...] = y1
    ms[...] = s

    keys = _sortkey(s)

    def count_ge(t):
        return jnp.sum((keys >= t).astype(jnp.float32))

    p0 = jnp.where(count_ge(0) >= _K, jnp.int32(0), jnp.int32(_IMIN))

    def bit_body(b, p):
        cand = p | jnp.left_shift(jnp.int32(1), 30 - b)
        return jnp.where(count_ge(cand) >= _K, cand, p)

    thr = lax.fori_loop(0, 31, bit_body, p0)

    gt = (keys > thr).astype(jnp.float32)
    eq = (keys == thr).astype(jnp.float32)
    ci = lax.broadcasted_iota(jnp.int32, (128, 128), 0)
    cj = lax.broadcasted_iota(jnp.int32, (128, 128), 1)
    lt128 = (ci < cj).astype(jnp.float32)
    ri = lax.broadcasted_iota(jnp.int32, (_ROWS, _ROWS), 0)
    rj = lax.broadcasted_iota(jnp.int32, (_ROWS, _ROWS), 1)
    sl160 = (rj < ri).astype(jnp.float32)

    def excl_prefix(m):
        lane = jnp.dot(m, lt128, preferred_element_type=jnp.float32)
        rows = jnp.sum(m, axis=1, keepdims=True)
        off = jnp.dot(sl160, rows, preferred_element_type=jnp.float32)
        return off + lane

    pos_gt = excl_prefix(gt)
    pos_eq = excl_prefix(eq)
    g_total = jnp.sum(gt)
    take_eq = (eq > 0) & (g_total + pos_eq < _K)
    p_f = jnp.where(gt > 0, pos_gt,
                    jnp.where(take_eq, g_total + pos_eq,
                              jnp.float32(_KPAD - 1)))
    pos[...] = p_f.astype(jnp.int32)


def _prep(cols):
    shp = jax.ShapeDtypeStruct((_ROWS, 128), jnp.float32)
    return pl.pallas_call(
        _prep_body,
        out_shape=[shp] * 5 + [jax.ShapeDtypeStruct((_ROWS, 128), jnp.int32)],
    )(*cols)


_sc_mesh = plsc.VectorSubcoreMesh(core_axis_name="c", subcore_axis_name="s")


@functools.partial(
    pl.kernel,
    mesh=_sc_mesh,
    out_type=jax.ShapeDtypeStruct((_KPAD, 8), jnp.float32),
    scratch_types=[
        pltpu.VMEM((_ROWS, 128), jnp.int32),
        pltpu.VMEM((_RPW * 128, 8), jnp.float32),
    ],
    compiler_params=pltpu.CompilerParams(use_tc_tiling_on_sc=False),
)
def _sc_compact(pos_hbm, packed_hbm, out_hbm, pidx, rows):
    wid = lax.axis_index("s") * 2 + lax.axis_index("c")
    pltpu.sync_copy(pos_hbm, pidx)
    pltpu.sync_copy(packed_hbm.at[pl.ds(wid * _RPW * 128, _RPW * 128)], rows)
    for c in range(_RPW):
        pltpu.sync_copy(rows.at[pl.ds(c * 128, 128)],
                        out_hbm.at[pidx.at[wid * _RPW + c]])


def _nms_body(x0c, y0c, x1c, y1c, scc, x0r, y0r, x1r, y1r, scr,
              pos_out, ox0, oy0, ox1, oy1, osc, m_scr, p_scr):
    cid = lax.broadcasted_iota(jnp.int32, (1, _KPAD), 1)
    vcol = cid < _K
    sr = jnp.where(vcol, scr[...], -jnp.inf)
    xr0 = jnp.where(vcol, x0r[...], 0.0)
    yr0 = jnp.where(vcol, y0r[...], 0.0)
    xr1 = jnp.where(vcol, x1r[...], 0.0)
    yr1 = jnp.where(vcol, y1r[...], 0.0)
    kcol_r = jnp.where(vcol, _sortkey(sr), _IMIN)
    area_r = (xr1 - xr0) * (yr1 - yr0)

    cid8 = lax.broadcasted_iota(jnp.int32, (256, _KPAD), 1)
    for t in range(_KPAD // 256):
        sl = pl.ds(t * 256, 256)
        rid = t * 256 + lax.broadcasted_iota(jnp.int32, (256, _KPAD), 0)
        vrow = rid < _K
        tx0 = jnp.where(vrow, x0c[sl, :], 0.0)
        ty0 = jnp.where(vrow, y0c[sl, :], 0.0)
        tx1 = jnp.where(vrow, x1c[sl, :], 0.0)
        ty1 = jnp.where(vrow, y1c[sl, :], 0.0)
        ts = jnp.where(vrow, scc[sl, :], -jnp.inf)
        tk = jnp.where(vrow, _sortkey(ts), _IMIN)
        area_c = (tx1 - tx0) * (ty1 - ty0)
        wx = jnp.clip(jnp.minimum(tx1, xr1) - jnp.maximum(tx0, xr0), 0.0, None)
        wy = jnp.clip(jnp.minimum(ty1, yr1) - jnp.maximum(ty0, yr0), 0.0, None)
        inter = wx * wy
        iou = inter / (area_c + area_r - inter + 1e-9)
        prio = (tk > kcol_r) | ((tk == kcol_r) & (rid < cid8))
        p_scr[sl, :] = prio.astype(jnp.bfloat16)
        m_scr[sl, :] = (prio & (iou > _THRESH)).astype(jnp.bfloat16)

    def cond(carry):
        return carry[1]

    def body(carry):
        k, _ = carry
        kb = jnp.broadcast_to(k, (8, _KPAD)).astype(jnp.bfloat16)
        supp = jnp.dot(kb, m_scr[...], preferred_element_type=jnp.float32)
        k_new = jnp.where(supp[0:1, :] > 0.0, 0.0, 1.0)
        return k_new, jnp.any(k_new != k)

    k0 = jnp.ones((1, _KPAD), jnp.float32)
    k_fin, _ = lax.while_loop(cond, body, (k0, True))
    kept = (k_fin > 0.0) & vcol

    keptb = jnp.broadcast_to(kept.astype(jnp.float32),
                             (8, _KPAD)).astype(jnp.bfloat16)
    rank = jnp.dot(keptb, p_scr[...],
                   preferred_element_type=jnp.float32)[0:1, :]
    ok = kept & (sr > -jnp.inf) & (rank < _OUT)
    m_cnt = jnp.sum(ok.astype(jnp.float32))

    for t in range(_KPAD // 256):
        sl = pl.ds(t * 256, 256)
        rid = t * 256 + lax.broadcasted_iota(jnp.int32, (256, _KPAD), 0)
        m_scr[sl, :] = (rid < cid8).astype(jnp.bfloat16)
    nokb = jnp.broadcast_to((~ok).astype(jnp.float32),
                            (8, _KPAD)).astype(jnp.bfloat16)
    nrank = jnp.dot(nokb, m_scr[...],
                    preferred_element_type=jnp.float32)[0:1, :]

    pos_out[...] = jnp.where(ok, rank, m_cnt + nrank).astype(jnp.int32)
    ox0[...] = jnp.where(ok, xr0, 0.0)
    oy0[...] = jnp.where(ok, yr0, 0.0)
    ox1[...] = jnp.where(ok, xr1, 0.0)
    oy1[...] = jnp.where(ok, yr1, 0.0)
    osc[...] = jnp.where(ok, sr, 0.0)


def _nms(sel):
    colrefs = [sel[:, i:i + 1] for i in range(5)]
    rowrefs = [sel[:, i].reshape(1, _KPAD) for i in range(5)]
    row = jax.ShapeDtypeStruct((1, _KPAD), jnp.float32)
    return pl.pallas_call(
        _nms_body,
        out_shape=[jax.ShapeDtypeStruct((1, _KPAD), jnp.int32)] + [row] * 5,
        scratch_shapes=[pltpu.VMEM((_KPAD, _KPAD), jnp.bfloat16),
                        pltpu.VMEM((_KPAD, _KPAD), jnp.bfloat16)],
    )(*colrefs, *rowrefs)


@functools.partial(
    pl.kernel,
    mesh=_sc_mesh,
    out_type=jax.ShapeDtypeStruct((_KPAD, 8), jnp.float32),
    scratch_types=[
        pltpu.VMEM((_KPAD // 128, 128), jnp.int32),
        pltpu.VMEM((128, 8), jnp.float32),
    ],
    compiler_params=pltpu.CompilerParams(use_tc_tiling_on_sc=False),
)
def _sc_emit(pos_hbm, rows_hbm, out_hbm, pidx, rows):
    wid = lax.axis_index("s") * 2 + lax.axis_index("c")

    @pl.when(wid < _KPAD // 128)
    def _():
        pltpu.sync_copy(pos_hbm, pidx)
        pltpu.sync_copy(rows_hbm.at[pl.ds(wid * 128, 128)], rows)
        pltpu.sync_copy(rows, out_hbm.at[pidx.at[wid]])


def kernel(anchors, regressions, scores):
    pad = _NPAD - _N

    def col(x):
        return jnp.pad(x, (0, pad)).reshape(_ROWS, 128)

    cols = ([col(anchors[:, i]) for i in range(4)]
            + [col(regressions[:, i]) for i in range(4)]
            + [col(scores)])
    bx0, by0, bx1, by1, ms, pos = _prep(cols)

    packed = jnp.stack(
        [bx0.reshape(-1), by0.reshape(-1), bx1.reshape(-1), by1.reshape(-1),
         ms.reshape(-1)] + [jnp.zeros((_NPAD,), jnp.float32)] * 3, axis=1)
    sel = _sc_compact(pos, packed)

    pos2, mx0, my0, mx1, my1, msc = _nms(sel)

    rows2 = jnp.stack(
        [mx0.reshape(-1), my0.reshape(-1), mx1.reshape(-1), my1.reshape(-1),
         msc.reshape(-1)] + [jnp.zeros((_KPAD,), jnp.float32)] * 3, axis=1)
    outp = _sc_emit(pos2.reshape(_KPAD // 128, 128), rows2)
    return outp[:_OUT, :5]


# speedup vs baseline: 1.7212x; 1.7212x over previous
import functools
import math

import jax
from jax import lax
import jax.numpy as jnp
from jax.experimental import pallas as pl
from jax.experimental.pallas import tpu as pltpu
from jax.experimental.pallas import tpu_sc as plsc

_IMG = 800.0
_N = 20000
_NPAD = 20480
_ROWS = _NPAD // 128
_K = 2000
_KPAD = 2048
_OUT = 1000
_THRESH = 0.7
_MIN_SIZE = 0.001
_LOG_MAX = math.log(1000.0 / 16)
_IMIN = -(2 ** 31)
_NWORK = 32
_RPW = _ROWS // _NWORK


def _sortkey(x):
    b = lax.bitcast_convert_type(x, jnp.int32)
    return jnp.where(b >= 0, b, jnp.bitwise_xor(~b, jnp.int32(_IMIN)))


def _prep_body(ax0, ay0, ax1, ay1, rdx, rdy, rdw, rdh, sc,
               bx0, by0, bx1, by1, ms, pos):
    width = ax1[...] - ax0[...]
    height = ay1[...] - ay0[...]
    cx = ax0[...] + width / 2
    cy = ay0[...] + height / 2
    dw = jnp.minimum(rdw[...], _LOG_MAX)
    dh = jnp.minimum(rdh[...], _LOG_MAX)
    px = cx + rdx[...] * width
    py = cy + rdy[...] * height
    pw = jnp.exp(dw) * width
    ph = jnp.exp(dh) * height
    x0 = jnp.clip(px - 0.5 * pw, 0.0, _IMG)
    y0 = jnp.clip(py - 0.5 * ph, 0.0, _IMG)
    x1 = jnp.clip(px + 0.5 * pw, 0.0, _IMG)
    y1 = jnp.clip(py + 0.5 * ph, 0.0, _IMG)
    valid = ((x1 - x0) >= _MIN_SIZE) & ((y1 - y0) >= _MIN_SIZE)
    s = jnp.where(valid, sc[...], -jnp.inf)
    bx0[...] = x0
    by0[...] = y0
    bx1[...] = x1
    by1[...] = y1
    ms[...] = s

    keys = _sortkey(s)

    def count_ge(t):
        return jnp.sum((keys >= t).astype(jnp.float32))

    p0 = jnp.where(count_ge(0) >= _K, jnp.int32(0), jnp.int32(_IMIN))

    def bit_body(b, p):
        cand = p | jnp.left_shift(jnp.int32(1), 30 - b)
        return jnp.where(count_ge(cand) >= _K, cand, p)

    thr = lax.fori_loop(0, 31, bit_body, p0)

    gt = (keys > thr).astype(jnp.float32)
    eq = (keys == thr).astype(jnp.float32)
    ci = lax.broadcasted_iota(jnp.int32, (128, 128), 0)
    cj = lax.broadcasted_iota(jnp.int32, (128, 128), 1)
    lt128 = (ci < cj).astype(jnp.float32)
    ri = lax.broadcasted_iota(jnp.int32, (_ROWS, _ROWS), 0)
    rj = lax.broadcasted_iota(jnp.int32, (_ROWS, _ROWS), 1)
    sl160 = (rj < ri).astype(jnp.float32)

    def excl_prefix(m):
        lane = jnp.dot(m, lt128, preferred_element_type=jnp.float32)
        rows = jnp.sum(m, axis=1, keepdims=True)
        off = jnp.dot(sl160, rows, preferred_element_type=jnp.float32)
        return off + lane

    pos_gt = excl_prefix(gt)
    pos_eq = excl_prefix(eq)
    g_total = jnp.sum(gt)
    take_eq = (eq > 0) & (g_total + pos_eq < _K)
    flat = (lax.broadcasted_iota(jnp.int32, (_ROWS, 128), 0) * 128
            + lax.broadcasted_iota(jnp.int32, (_ROWS, 128), 1))
    trash = (_KPAD + flat).astype(jnp.float32)
    p_f = jnp.where(gt > 0, pos_gt,
                    jnp.where(take_eq, g_total + pos_eq, trash))
    pos[...] = p_f.astype(jnp.int32)


def _prep(cols):
    shp = jax.ShapeDtypeStruct((_ROWS, 128), jnp.float32)
    return pl.pallas_call(
        _prep_body,
        out_shape=[shp] * 5 + [jax.ShapeDtypeStruct((_ROWS, 128), jnp.int32)],
    )(*cols)


_sc_mesh = plsc.VectorSubcoreMesh(core_axis_name="c", subcore_axis_name="s")


@functools.partial(
    pl.kernel,
    mesh=_sc_mesh,
    out_type=jax.ShapeDtypeStruct((_KPAD + _NPAD, 8), jnp.float32),
    scratch_types=[
        pltpu.VMEM((_ROWS, 128), jnp.int32),
        pltpu.VMEM((_RPW * 128, 8), jnp.float32),
    ],
    compiler_params=pltpu.CompilerParams(use_tc_tiling_on_sc=False),
)
def _sc_compact(pos_hbm, packed_hbm, out_hbm, pidx, rows):
    wid = lax.axis_index("s") * 2 + lax.axis_index("c")
    pltpu.sync_copy(pos_hbm, pidx)
    pltpu.sync_copy(packed_hbm.at[pl.ds(wid * _RPW * 128, _RPW * 128)], rows)
    for c in range(_RPW):
        pltpu.sync_copy(rows.at[pl.ds(c * 128, 128)],
                        out_hbm.at[pidx.at[wid * _RPW + c]])


def _nms_body(x0c, y0c, x1c, y1c, scc, x0r, y0r, x1r, y1r, scr,
              pos_out, ox0, oy0, ox1, oy1, osc, m_scr, p_scr):
    cid = lax.broadcasted_iota(jnp.int32, (1, _KPAD), 1)
    vcol = cid < _K
    sr = jnp.where(vcol, scr[...], -jnp.inf)
    xr0 = jnp.where(vcol, x0r[...], 0.0)
    yr0 = jnp.where(vcol, y0r[...], 0.0)
    xr1 = jnp.where(vcol, x1r[...], 0.0)
    yr1 = jnp.where(vcol, y1r[...], 0.0)
    kcol_r = jnp.where(vcol, _sortkey(sr), _IMIN)
    area_r = (xr1 - xr0) * (yr1 - yr0)

    cid8 = lax.broadcasted_iota(jnp.int32, (256, _KPAD), 1)
    for t in range(_KPAD // 256):
        sl = pl.ds(t * 256, 256)
        rid = t * 256 + lax.broadcasted_iota(jnp.int32, (256, _KPAD), 0)
        vrow = rid < _K
        tx0 = jnp.where(vrow, x0c[sl, :], 0.0)
        ty0 = jnp.where(vrow, y0c[sl, :], 0.0)
        tx1 = jnp.where(vrow, x1c[sl, :], 0.0)
        ty1 = jnp.where(vrow, y1c[sl, :], 0.0)
        ts = jnp.where(vrow, scc[sl, :], -jnp.inf)
        tk = jnp.where(vrow, _sortkey(ts), _IMIN)
        area_c = (tx1 - tx0) * (ty1 - ty0)
        wx = jnp.clip(jnp.minimum(tx1, xr1) - jnp.maximum(tx0, xr0), 0.0, None)
        wy = jnp.clip(jnp.minimum(ty1, yr1) - jnp.maximum(ty0, yr0), 0.0, None)
        inter = wx * wy
        iou = inter / (area_c + area_r - inter + 1e-9)
        prio = (tk > kcol_r) | ((tk == kcol_r) & (rid < cid8))
        p_scr[sl, :] = prio.astype(jnp.bfloat16)
        m_scr[sl, :] = (prio & (iou > _THRESH)).astype(jnp.bfloat16)

    def cond(carry):
        return carry[1]

    def body(carry):
        k, _ = carry
        kb = jnp.broadcast_to(k, (8, _KPAD)).astype(jnp.bfloat16)
        supp = jnp.dot(kb, m_scr[...], preferred_element_type=jnp.float32)
        k_new = jnp.where(supp[0:1, :] > 0.0, 0.0, 1.0)
        return k_new, jnp.any(k_new != k)

    k0 = jnp.ones((1, _KPAD), jnp.float32)
    k_fin, _ = lax.while_loop(cond, body, (k0, True))
    kept = (k_fin > 0.0) & vcol

    keptb = jnp.broadcast_to(kept.astype(jnp.float32),
                             (8, _KPAD)).astype(jnp.bfloat16)
    rank = jnp.dot(keptb, p_scr[...],
                   preferred_element_type=jnp.float32)[0:1, :]
    ok = kept & (sr > -jnp.inf) & (rank < _OUT)
    m_cnt = jnp.sum(ok.astype(jnp.float32))

    for t in range(_KPAD // 256):
        sl = pl.ds(t * 256, 256)
        rid = t * 256 + lax.broadcasted_iota(jnp.int32, (256, _KPAD), 0)
        m_scr[sl, :] = (rid < cid8).astype(jnp.bfloat16)
    nokb = jnp.broadcast_to((~ok).astype(jnp.float32),
                            (8, _KPAD)).astype(jnp.bfloat16)
    nrank = jnp.dot(nokb, m_scr[...],
                    preferred_element_type=jnp.float32)[0:1, :]

    pos_out[...] = jnp.where(ok, rank, m_cnt + nrank).astype(jnp.int32)
    ox0[...] = jnp.where(ok, xr0, 0.0)
    oy0[...] = jnp.where(ok, yr0, 0.0)
    ox1[...] = jnp.where(ok, xr1, 0.0)
    oy1[...] = jnp.where(ok, yr1, 0.0)
    osc[...] = jnp.where(ok, sr, 0.0)


def _nms(sel):
    colrefs = [sel[:, i:i + 1] for i in range(5)]
    rowrefs = [sel[:, i].reshape(1, _KPAD) for i in range(5)]
    row = jax.ShapeDtypeStruct((1, _KPAD), jnp.float32)
    return pl.pallas_call(
        _nms_body,
        out_shape=[jax.ShapeDtypeStruct((1, _KPAD), jnp.int32)] + [row] * 5,
        scratch_shapes=[pltpu.VMEM((_KPAD, _KPAD), jnp.bfloat16),
                        pltpu.VMEM((_KPAD, _KPAD), jnp.bfloat16)],
    )(*colrefs, *rowrefs)


@functools.partial(
    pl.kernel,
    mesh=_sc_mesh,
    out_type=jax.ShapeDtypeStruct((_KPAD, 8), jnp.float32),
    scratch_types=[
        pltpu.VMEM((_KPAD // 128, 128), jnp.int32),
        pltpu.VMEM((128, 8), jnp.float32),
    ],
    compiler_params=pltpu.CompilerParams(use_tc_tiling_on_sc=False),
)
def _sc_emit(pos_hbm, rows_hbm, out_hbm, pidx, rows):
    wid = lax.axis_index("s") * 2 + lax.axis_index("c")

    @pl.when(wid < _KPAD // 128)
    def _():
        pltpu.sync_copy(pos_hbm, pidx)
        pltpu.sync_copy(rows_hbm.at[pl.ds(wid * 128, 128)], rows)
        pltpu.sync_copy(rows, out_hbm.at[pidx.at[wid]])


def kernel(anchors, regressions, scores):
    pad = _NPAD - _N

    def col(x):
        return jnp.pad(x, (0, pad)).reshape(_ROWS, 128)

    cols = ([col(anchors[:, i]) for i in range(4)]
            + [col(regressions[:, i]) for i in range(4)]
            + [col(scores)])
    bx0, by0, bx1, by1, ms, pos = _prep(cols)

    packed = jnp.stack(
        [bx0.reshape(-1), by0.reshape(-1), bx1.reshape(-1), by1.reshape(-1),
         ms.reshape(-1)] + [jnp.zeros((_NPAD,), jnp.float32)] * 3, axis=1)
    sel = _sc_compact(pos, packed)[:_KPAD]

    pos2, mx0, my0, mx1, my1, msc = _nms(sel)

    rows2 = jnp.stack(
        [mx0.reshape(-1), my0.reshape(-1), mx1.reshape(-1), my1.reshape(-1),
         msc.reshape(-1)] + [jnp.zeros((_KPAD,), jnp.float32)] * 3, axis=1)
    outp = _sc_emit(pos2.reshape(_KPAD // 128, 128), rows2)
    return outp[:_OUT, :5]


# speedup vs baseline: 1.7644x; 1.0251x over previous
import functools
import math

import jax
from jax import lax
import jax.numpy as jnp
from jax.experimental import pallas as pl
from jax.experimental.pallas import tpu as pltpu
from jax.experimental.pallas import tpu_sc as plsc

_IMG = 800.0
_N = 20000
_NPAD = 20480
_ROWS = _NPAD // 128
_K = 2000
_KPAD = 2048
_OUT = 1000
_THRESH = 0.7
_MIN_SIZE = 0.001
_LOG_MAX = math.log(1000.0 / 16)
_IMIN = -(2 ** 31)
_NWORK = 32
_RPW = _ROWS // _NWORK


def _sortkey(x):
    b = lax.bitcast_convert_type(x, jnp.int32)
    return jnp.where(b >= 0, b, jnp.bitwise_xor(~b, jnp.int32(_IMIN)))


def _prep_body(ax0, ay0, ax1, ay1, rdx, rdy, rdw, rdh, sc,
               bx0, by0, bx1, by1, ms, pos):
    width = ax1[...] - ax0[...]
    height = ay1[...] - ay0[...]
    cx = ax0[...] + width / 2
    cy = ay0[...] + height / 2
    dw = jnp.minimum(rdw[...], _LOG_MAX)
    dh = jnp.minimum(rdh[...], _LOG_MAX)
    px = cx + rdx[...] * width
    py = cy + rdy[...] * height
    pw = jnp.exp(dw) * width
    ph = jnp.exp(dh) * height
    x0 = jnp.clip(px - 0.5 * pw, 0.0, _IMG)
    y0 = jnp.clip(py - 0.5 * ph, 0.0, _IMG)
    x1 = jnp.clip(px + 0.5 * pw, 0.0, _IMG)
    y1 = jnp.clip(py + 0.5 * ph, 0.0, _IMG)
    valid = ((x1 - x0) >= _MIN_SIZE) & ((y1 - y0) >= _MIN_SIZE)
    s = jnp.where(valid, sc[...], -jnp.inf)
    bx0[...] = x0
    by0[...] = y0
    bx1[...] = x1
    by1[...] = y1
    ms[...] = s

    keys = _sortkey(s)

    def count_ge(t):
        return jnp.sum((keys >= t).astype(jnp.float32))

    p0 = jnp.where(count_ge(0) >= _K, jnp.int32(0), jnp.int32(_IMIN))

    def bit_body(b, p):
        cand = p | jnp.left_shift(jnp.int32(1), 30 - b)
        return jnp.where(count_ge(cand) >= _K, cand, p)

    thr = lax.fori_loop(0, 31, bit_body, p0)

    gt = (keys > thr).astype(jnp.float32)
    eq = (keys == thr).astype(jnp.float32)
    ci = lax.broadcasted_iota(jnp.int32, (128, 128), 0)
    cj = lax.broadcasted_iota(jnp.int32, (128, 128), 1)
    lt128 = (ci < cj).astype(jnp.float32)
    ri = lax.broadcasted_iota(jnp.int32, (_ROWS, _ROWS), 0)
    rj = lax.broadcasted_iota(jnp.int32, (_ROWS, _ROWS), 1)
    sl160 = (rj < ri).astype(jnp.float32)

    def excl_prefix(m):
        lane = jnp.dot(m, lt128, preferred_element_type=jnp.float32)
        rows = jnp.sum(m, axis=1, keepdims=True)
        off = jnp.dot(sl160, rows, preferred_element_type=jnp.float32)
        return off + lane

    pos_gt = excl_prefix(gt)
    pos_eq = excl_prefix(eq)
    g_total = jnp.sum(gt)
    take_eq = (eq > 0) & (g_total + pos_eq < _K)
    flat = (lax.broadcasted_iota(jnp.int32, (_ROWS, 128), 0) * 128
            + lax.broadcasted_iota(jnp.int32, (_ROWS, 128), 1))
    trash = (_KPAD + flat).astype(jnp.float32)
    p_f = jnp.where(gt > 0, pos_gt,
                    jnp.where(take_eq, g_total + pos_eq, trash))
    pos[...] = p_f.astype(jnp.int32)


def _prep(cols):
    shp = jax.ShapeDtypeStruct((_ROWS, 128), jnp.float32)
    return pl.pallas_call(
        _prep_body,
        out_shape=[shp] * 5 + [jax.ShapeDtypeStruct((_ROWS, 128), jnp.int32)],
    )(*cols)


_sc_mesh = plsc.VectorSubcoreMesh(core_axis_name="c", subcore_axis_name="s")


@functools.partial(
    pl.kernel,
    mesh=_sc_mesh,
    out_type=jax.ShapeDtypeStruct((_KPAD + _NPAD, 8), jnp.float32),
    scratch_types=[
        pltpu.VMEM((16, 128), jnp.int32),
        pltpu.VMEM((_RPW * 128, 8), jnp.float32),
    ],
    compiler_params=pltpu.CompilerParams(use_tc_tiling_on_sc=False),
)
def _sc_compact(pos_hbm, packed_hbm, out_hbm, pidx, rows):
    wid = lax.axis_index("s") * 2 + lax.axis_index("c")
    win = pl.multiple_of(jnp.minimum((wid * _RPW // 8) * 8, _ROWS - 16), 8)
    pltpu.sync_copy(pos_hbm.at[pl.ds(win, 16)], pidx)
    pltpu.sync_copy(packed_hbm.at[pl.ds(wid * _RPW * 128, _RPW * 128)], rows)
    for c in range(_RPW):
        pltpu.sync_copy(rows.at[pl.ds(c * 128, 128)],
                        out_hbm.at[pidx.at[wid * _RPW + c - win]])


def _nms_body(x0c, y0c, x1c, y1c, scc, x0r, y0r, x1r, y1r, scr,
              pos_out, ox0, oy0, ox1, oy1, osc, m_scr, p_scr):
    cid = lax.broadcasted_iota(jnp.int32, (1, _KPAD), 1)
    vcol = cid < _K
    sr = jnp.where(vcol, scr[...], -jnp.inf)
    xr0 = jnp.where(vcol, x0r[...], 0.0)
    yr0 = jnp.where(vcol, y0r[...], 0.0)
    xr1 = jnp.where(vcol, x1r[...], 0.0)
    yr1 = jnp.where(vcol, y1r[...], 0.0)
    kcol_r = jnp.where(vcol, _sortkey(sr), _IMIN)
    area_r = (xr1 - xr0) * (yr1 - yr0)

    cid8 = lax.broadcasted_iota(jnp.int32, (256, _KPAD), 1)
    for t in range(_KPAD // 256):
        sl = pl.ds(t * 256, 256)
        rid = t * 256 + lax.broadcasted_iota(jnp.int32, (256, _KPAD), 0)
        rid1 = t * 256 + lax.broadcasted_iota(jnp.int32, (256, 1), 0)
        vrow1 = rid1 < _K
        tx0 = jnp.where(vrow1, x0c[sl, :], 0.0)
        ty0 = jnp.where(vrow1, y0c[sl, :], 0.0)
        tx1 = jnp.where(vrow1, x1c[sl, :], 0.0)
        ty1 = jnp.where(vrow1, y1c[sl, :], 0.0)
        ts = jnp.where(vrow1, scc[sl, :], -jnp.inf)
        tk = jnp.where(vrow1, _sortkey(ts), _IMIN)
        area_c = (tx1 - tx0) * (ty1 - ty0)
        wx = jnp.clip(jnp.minimum(tx1, xr1) - jnp.maximum(tx0, xr0), 0.0, None)
        wy = jnp.clip(jnp.minimum(ty1, yr1) - jnp.maximum(ty0, yr0), 0.0, None)
        inter = wx * wy
        iou = inter / (area_c + area_r - inter + 1e-9)
        prio = (tk > kcol_r) | ((tk == kcol_r) & (rid < cid8))
        p_scr[sl, :] = prio.astype(jnp.bfloat16)
        m_scr[sl, :] = (prio & (iou > _THRESH)).astype(jnp.bfloat16)

    def cond(carry):
        return carry[1]

    def body(carry):
        k, _ = carry
        kb = jnp.broadcast_to(k, (8, _KPAD)).astype(jnp.bfloat16)
        supp = jnp.dot(kb, m_scr[...], preferred_element_type=jnp.float32)
        k_new = jnp.where(supp[0:1, :] > 0.0, 0.0, 1.0)
        return k_new, jnp.any(k_new != k)

    k0 = jnp.ones((1, _KPAD), jnp.float32)
    k_fin, _ = lax.while_loop(cond, body, (k0, True))
    kept = (k_fin > 0.0) & vcol

    keptb = jnp.broadcast_to(kept.astype(jnp.float32),
                             (8, _KPAD)).astype(jnp.bfloat16)
    rank = jnp.dot(keptb, p_scr[...],
                   preferred_element_type=jnp.float32)[0:1, :]
    ok = kept & (sr > -jnp.inf) & (rank < _OUT)
    m_cnt = jnp.sum(ok.astype(jnp.float32))

    for t in range(_KPAD // 256):
        sl = pl.ds(t * 256, 256)
        rid = t * 256 + lax.broadcasted_iota(jnp.int32, (256, _KPAD), 0)
        m_scr[sl, :] = (rid < cid8).astype(jnp.bfloat16)
    nokb = jnp.broadcast_to((~ok).astype(jnp.float32),
                            (8, _KPAD)).astype(jnp.bfloat16)
    nrank = jnp.dot(nokb, m_scr[...],
                    preferred_element_type=jnp.float32)[0:1, :]

    pos_out[...] = jnp.where(ok, rank, m_cnt + nrank).astype(jnp.int32)
    ox0[...] = jnp.where(ok, xr0, 0.0)
    oy0[...] = jnp.where(ok, yr0, 0.0)
    ox1[...] = jnp.where(ok, xr1, 0.0)
    oy1[...] = jnp.where(ok, yr1, 0.0)
    osc[...] = jnp.where(ok, sr, 0.0)


def _nms(sel):
    colrefs = [sel[:, i:i + 1] for i in range(5)]
    rowrefs = [sel[:, i].reshape(1, _KPAD) for i in range(5)]
    row = jax.ShapeDtypeStruct((1, _KPAD), jnp.float32)
    return pl.pallas_call(
        _nms_body,
        out_shape=[jax.ShapeDtypeStruct((1, _KPAD), jnp.int32)] + [row] * 5,
        scratch_shapes=[pltpu.VMEM((_KPAD, _KPAD), jnp.bfloat16),
                        pltpu.VMEM((_KPAD, _KPAD), jnp.bfloat16)],
    )(*colrefs, *rowrefs)


@functools.partial(
    pl.kernel,
    mesh=_sc_mesh,
    out_type=jax.ShapeDtypeStruct((_KPAD, 8), jnp.float32),
    scratch_types=[
        pltpu.VMEM((_KPAD // 128, 128), jnp.int32),
        pltpu.VMEM((128, 8), jnp.float32),
    ],
    compiler_params=pltpu.CompilerParams(use_tc_tiling_on_sc=False),
)
def _sc_emit(pos_hbm, rows_hbm, out_hbm, pidx, rows):
    wid = lax.axis_index("s") * 2 + lax.axis_index("c")

    @pl.when(wid < _KPAD // 128)
    def _():
        pltpu.sync_copy(pos_hbm, pidx)
        pltpu.sync_copy(rows_hbm.at[pl.ds(wid * 128, 128)], rows)
        pltpu.sync_copy(rows, out_hbm.at[pidx.at[wid]])


def kernel(anchors, regressions, scores):
    pad = _NPAD - _N

    def col(x):
        return jnp.pad(x, (0, pad)).reshape(_ROWS, 128)

    cols = ([col(anchors[:, i]) for i in range(4)]
            + [col(regressions[:, i]) for i in range(4)]
            + [col(scores)])
    bx0, by0, bx1, by1, ms, pos = _prep(cols)

    packed = jnp.stack(
        [bx0.reshape(-1), by0.reshape(-1), bx1.reshape(-1), by1.reshape(-1),
         ms.reshape(-1)] + [jnp.zeros((_NPAD,), jnp.float32)] * 3, axis=1)
    sel = _sc_compact(pos, packed)[:_KPAD]

    pos2, mx0, my0, mx1, my1, msc = _nms(sel)

    rows2 = jnp.stack(
        [mx0.reshape(-1), my0.reshape(-1), mx1.reshape(-1), my1.reshape(-1),
         msc.reshape(-1)] + [jnp.zeros((_KPAD,), jnp.float32)] * 3, axis=1)
    outp = _sc_emit(pos2.reshape(_KPAD // 128, 128), rows2)
    return outp[:_OUT, :5]
